# baseline (speedup 1.0000x reference)
import functools

import jax
import jax.numpy as jnp
from jax import lax
from jax.experimental import pallas as pl
from jax.experimental.pallas import tpu as pltpu
from jax.experimental.pallas import tpu_sc as plsc

D = 128
NT = 16
K = 128
ZPT = 88
BCH = 16


def _norm_rows(x):
    n = jnp.sqrt(jnp.sum(x * x, axis=1, keepdims=True))
    return x / jnp.maximum(n, 1e-12)


def _leaky(x):
    return jnp.where(x >= 0, x, 0.01 * x)


def _tc_pre_body(pref_ref, feat_ref, mlp_w_ref, mlp_b_ref, p0_ref, fm_ref):
    p0 = _norm_rows(pref_ref[...])
    h = lax.dot_general(feat_ref[...], mlp_w_ref[...],
                        (((1,), (1,)), ((), ())),
                        preferred_element_type=jnp.float32) + mlp_b_ref[...]
    p0_ref[...] = p0
    fm_ref[...] = _norm_rows(_leaky(h))


def _tc_pre(preference, features, mlp_w, mlp_b):
    n_u, n_i = preference.shape[0], features.shape[0]
    return pl.pallas_call(
        _tc_pre_body,
        out_shape=[
            jax.ShapeDtypeStruct((n_u, D), jnp.float32),
            jax.ShapeDtypeStruct((n_i, D), jnp.float32),
        ],
    )(preference, features, mlp_w, mlp_b.reshape(1, D))


def _tc_mid_body(xu_ref, hu_ref, w_ref, pref_ref):
    hcu = jnp.dot(hu_ref[...], w_ref[...], preferred_element_type=jnp.float32)
    pref_ref[...] = _norm_rows(xu_ref[...] + hcu)


def _tc_mid(xu, hu, w):
    return pl.pallas_call(
        _tc_mid_body,
        out_shape=jax.ShapeDtypeStruct(xu.shape, jnp.float32),
    )(xu, hu, w)


def _tc_fin_body(x2_ref, h_ref, w_ref, out_ref):
    hc = jnp.dot(h_ref[...], w_ref[...], preferred_element_type=jnp.float32)
    out_ref[...] = x2_ref[...] + _leaky(hc)


def _tc_fin(x2, h, w):
    return pl.pallas_call(
        _tc_fin_body,
        out_shape=jax.ShapeDtypeStruct(x2.shape, jnp.float32),
    )(x2, h, w)


@functools.lru_cache(maxsize=None)
def _sc_agg(nc, n_rows, zrow0):
    part = n_rows // 8
    acc_rows = part
    zpt = acc_rows // NT
    dpt = part // NT
    assert part % (NT * 8) == 0 and zpt % 8 == 0 and zpt <= ZPT
    mesh = plsc.VectorSubcoreMesh(core_axis_name="c", subcore_axis_name="s")

    @functools.partial(
        pl.kernel,
        out_type=jax.ShapeDtypeStruct((n_rows, D), jnp.float32),
        mesh=mesh,
        compiler_params=pltpu.CompilerParams(needs_layout_passes=False),
        scratch_types=[
            pltpu.VMEM((2, BCH, K), jnp.int32),
            pltpu.VMEM((2, BCH, K), jnp.int32),
            pltpu.VMEM((2, K, D), jnp.float32),
            pltpu.VMEM((ZPT, D), jnp.float32),
            pltpu.VMEM((ZPT, D), jnp.float32),
            pltpu.VMEM((2 * K,), jnp.int32),
            pltpu.VMEM((2 * K,), jnp.int32),
            pltpu.VMEM((2, K), jnp.int32),
            pltpu.VMEM((2, K), jnp.int32),
            pltpu.VMEM_SHARED((acc_rows, D), jnp.float32),
            pltpu.VMEM((16,), jnp.int32),
            pltpu.SemaphoreType.DMA,
            pltpu.SemaphoreType.DMA,
            pltpu.SemaphoreType.DMA,
        ],
    )
    def agg(x_hbm, src_hbm, dst_hbm, zero_hbm, ncs_hbm, out_hbm,
            sblk_v, dblk_v, rows_v, zbuf_v, stage_v, csrc_v, cdst_v,
            fsrc_v, fdst_v, acc, ncs_v, gsem, isem, ssem):
        cid = lax.axis_index("c")
        sid = lax.axis_index("s")
        pltpu.sync_copy(ncs_hbm, ncs_v)
        pltpu.sync_copy(zero_hbm, zbuf_v)
        mv = ncs_v[pl.ds(0, 16)]
        ncd = mv[0]
        it = mv[8]
        lane = lax.broadcasted_iota(jnp.int32, (16,), 0)

        def start_blk(buf, b0):
            pltpu.async_copy(src_hbm.at[it, sid, pl.ds(b0, BCH)],
                             sblk_v.at[buf], isem)
            pltpu.async_copy(dst_hbm.at[it, sid, pl.ds(b0, BCH)],
                             dblk_v.at[buf], isem)

        def wait_blk(buf, b0):
            pltpu.make_async_copy(src_hbm.at[it, sid, pl.ds(b0, BCH)],
                                  sblk_v.at[buf], isem).wait()
            pltpu.make_async_copy(dst_hbm.at[it, sid, pl.ds(b0, BCH)],
                                  dblk_v.at[buf], isem).wait()

        def wait_gather(b):
            pltpu.make_async_copy(x_hbm.at[fsrc_v.at[b]], rows_v.at[b],
                                  gsem).wait()

        def start_scatter(b):
            pltpu.async_copy(rows_v.at[b], acc.at[fdst_v.at[b]], ssem,
                             add=True)

        def wait_scatter(b):
            pltpu.make_async_copy(rows_v.at[b], acc.at[fdst_v.at[b]],
                                  ssem).wait()

        def flush_static(b, pend):
            @pl.when(pend >= 2)
            def _():
                wait_scatter(b)

            for j in range(K // 16):
                fsrc_v[b, pl.ds(16 * j, 16)] = csrc_v[pl.ds(16 * j, 16)]
                fdst_v[b, pl.ds(16 * j, 16)] = cdst_v[pl.ds(16 * j, 16)]
            @pl.when(pend >= 1)
            def _():
                wait_gather(1 - b)
                start_scatter(1 - b)

            pltpu.async_copy(x_hbm.at[fsrc_v.at[b]], rows_v.at[b], gsem)

        def flush(fb, pend, extra=None):
            for b in range(2):
                @pl.when(fb == b)
                def _(b=b):
                    flush_static(b, pend)
                    if extra is not None:
                        extra()

        def drain_all(fb, pend):
            for b in range(2):
                @pl.when(fb == b)
                def _(b=b):
                    wait_gather(b)
                    start_scatter(b)

                    @pl.when(pend >= 2)
                    def _():
                        wait_scatter(1 - b)

                    wait_scatter(b)

        def move_ring_down():
            for j in range(K // 16):
                csrc_v[pl.ds(16 * j, 16)] = csrc_v[pl.ds(K + 16 * j, 16)]
                cdst_v[pl.ds(16 * j, 16)] = cdst_v[pl.ds(K + 16 * j, 16)]

        z0 = sid * zpt
        pltpu.sync_copy(zbuf_v, acc.at[pl.ds(z0, zpt)])
        plsc.subcore_barrier()

        for p in range(4):
            lo = (4 * cid + p) * part

            def proc_block(buf, carry):
                def chunk(c, carry):
                    ptrv, fb, pend = carry
                    for j in range(K // 16):
                        d = dblk_v[buf, c, pl.ds(16 * j, 16)]
                        s = sblk_v[buf, c, pl.ds(16 * j, 16)]
                        dl = d - lo
                        m = dl.astype(jnp.uint32) < jnp.uint32(part)
                        m32 = m.astype(jnp.int32)
                        cs = plsc.cumsum(m32)
                        pos = ptrv + cs - m32
                        plsc.store_scatter(csrc_v, [pos], s, mask=m)
                        plsc.store_scatter(cdst_v, [pos], dl, mask=m)
                        ptrv = ptrv + plsc.all_reduce_population_count(m)
                    full = ptrv[0] >= K

                    @pl.when(full)
                    def _():
                        flush(fb, pend, extra=move_ring_down)

                    ptrv = jnp.where(full, ptrv - K, ptrv)
                    pend = jnp.where(full, jnp.minimum(pend + 1, 2), pend)
                    fb = jnp.where(full, 1 - fb, fb)
                    return ptrv, fb, pend

                return lax.fori_loop(0, BCH, chunk, carry, unroll=False)

            start_blk(0, 0)

            def blkpair(i, carry):
                b0 = 2 * i * BCH
                wait_blk(0, b0)
                start_blk(1, b0 + BCH)
                carry = proc_block(0, carry)
                wait_blk(1, b0 + BCH)

                @pl.when(b0 + 2 * BCH < ncd)
                def _():
                    start_blk(0, b0 + 2 * BCH)

                return proc_block(1, carry)

            ptrv, fb, pend = lax.fori_loop(
                0, ncd // (2 * BCH), blkpair,
                (jnp.zeros((16,), jnp.int32), jnp.int32(0), jnp.int32(0)),
                unroll=False)
            ptr = ptrv[0]
            for j in range(K // 16):
                idxv = ptr + 16 * j + lane
                plsc.store_scatter(csrc_v, [idxv],
                                   zrow0 + ((lane + 16 * j) & 15))
                plsc.store_scatter(cdst_v, [idxv],
                                   ((lane + 16 * j) * 8) & (part - 1))
            flush(fb, pend)
            pend = jnp.minimum(pend + 1, 2)
            drain_all(fb, pend)
            plsc.subcore_barrier()
            d0 = sid * dpt
            pltpu.sync_copy(acc.at[pl.ds(d0, dpt)], stage_v.at[pl.ds(0, dpt)])
            pltpu.sync_copy(stage_v.at[pl.ds(0, dpt)],
                            out_hbm.at[pl.ds(lo + d0, dpt)])
            if p < 3:
                pltpu.sync_copy(zbuf_v.at[pl.ds(0, dpt)],
                                acc.at[pl.ds(d0, dpt)])
            plsc.subcore_barrier()

    return agg


def _prep_edges(src, dst, n_nodes, ncmax):
    e = src.shape[0]
    per = NT * K
    nc = -(-(-(-e // per)) // (2 * BCH)) * (2 * BCH)
    pad = nc * per - e
    if pad:
        ar = jnp.arange(pad, dtype=jnp.int32)
        src = jnp.concatenate([src, (ar * 97) % n_nodes])
        dst = jnp.concatenate([dst, jnp.full((pad,), -1, jnp.int32)])
    src = src.reshape(NT, nc, K)
    dst = dst.reshape(NT, nc, K)
    if ncmax > nc:
        src = jnp.pad(src, ((0, 0), (0, ncmax - nc), (0, 0)))
        dst = jnp.pad(dst, ((0, 0), (0, ncmax - nc), (0, 0)), constant_values=-1)
    return src, dst, nc


def kernel(edge_index_drop, edge_index, features, preference, mlp_w, mlp_b,
           conv_w):
    n_user = preference.shape[0]
    n_nodes = n_user + features.shape[0]
    n_rows = -(-n_nodes // (4 * NT * 8)) * (4 * NT * 8)

    e1 = edge_index_drop.shape[1]
    e2 = 2 * edge_index.shape[1]
    ncmax = -(-(-(-max(e1, e2) // (NT * K))) // (2 * BCH)) * (2 * BCH)

    p0, fm = _tc_pre(preference, features, mlp_w, mlp_b)
    x0 = jnp.concatenate([p0, fm, jnp.zeros((16, D), jnp.float32)], axis=0)
    zeros = jnp.zeros((ZPT, D), jnp.float32)
    sc = _sc_agg(ncmax, n_rows, n_nodes)

    s1, d1, nc1 = _prep_edges(edge_index_drop[0], edge_index_drop[1],
                              n_nodes, ncmax)
    s2, d2, nc2 = _prep_edges(
        jnp.concatenate([edge_index[0], edge_index[1]]),
        jnp.concatenate([edge_index[1], edge_index[0]]),
        n_nodes, ncmax)
    src_st = jnp.stack([s1, s2])
    dst_st = jnp.stack([d1, d2])

    def body(i, carry):
        x, out = carry
        ncd = jnp.where(i == 0, nc1, nc2).astype(jnp.int32)
        meta = jnp.concatenate([jnp.full((8,), ncd, jnp.int32),
                                jnp.full((8,), i, jnp.int32)])
        h = sc(x, src_st, dst_st, zeros, meta)

        def mid(args):
            x, h = args
            pref = _tc_mid(x[:n_user], h[:n_user], conv_w)
            return jnp.concatenate([pref, x[n_user:]], axis=0), x[:n_nodes]

        def fin(args):
            x, h = args
            return x, _tc_fin(x[:n_nodes], h[:n_nodes], conv_w)

        return lax.cond(i == 0, mid, fin, (x, h))

    two = lax.optimization_barrier(jnp.int32(2))
    xf, out = lax.fori_loop(0, two, body, (x0, x0[:n_nodes]))
    return (out, xf[:n_user])


# speedup vs baseline: 1.0093x; 1.0093x over previous
import functools

import jax
import jax.numpy as jnp
from jax import lax
from jax.experimental import pallas as pl
from jax.experimental.pallas import tpu as pltpu
from jax.experimental.pallas import tpu_sc as plsc

D = 128
NT = 16
K = 128
ZPT = 88
BCH = 80


def _norm_rows(x):
    n = jnp.sqrt(jnp.sum(x * x, axis=1, keepdims=True))
    return x / jnp.maximum(n, 1e-12)


def _leaky(x):
    return jnp.where(x >= 0, x, 0.01 * x)


def _tc_pre_body(pref_ref, feat_ref, mlp_w_ref, mlp_b_ref, p0_ref, fm_ref):
    p0 = _norm_rows(pref_ref[...])
    h = lax.dot_general(feat_ref[...], mlp_w_ref[...],
                        (((1,), (1,)), ((), ())),
                        preferred_element_type=jnp.float32) + mlp_b_ref[...]
    p0_ref[...] = p0
    fm_ref[...] = _norm_rows(_leaky(h))


def _tc_pre(preference, features, mlp_w, mlp_b):
    n_u, n_i = preference.shape[0], features.shape[0]
    return pl.pallas_call(
        _tc_pre_body,
        out_shape=[
            jax.ShapeDtypeStruct((n_u, D), jnp.float32),
            jax.ShapeDtypeStruct((n_i, D), jnp.float32),
        ],
    )(preference, features, mlp_w, mlp_b.reshape(1, D))


def _tc_mid_body(xu_ref, hu_ref, w_ref, pref_ref):
    hcu = jnp.dot(hu_ref[...], w_ref[...], preferred_element_type=jnp.float32)
    pref_ref[...] = _norm_rows(xu_ref[...] + hcu)


def _tc_mid(xu, hu, w):
    return pl.pallas_call(
        _tc_mid_body,
        out_shape=jax.ShapeDtypeStruct(xu.shape, jnp.float32),
    )(xu, hu, w)


def _tc_fin_body(x2_ref, h_ref, w_ref, out_ref):
    hc = jnp.dot(h_ref[...], w_ref[...], preferred_element_type=jnp.float32)
    out_ref[...] = x2_ref[...] + _leaky(hc)


def _tc_fin(x2, h, w):
    return pl.pallas_call(
        _tc_fin_body,
        out_shape=jax.ShapeDtypeStruct(x2.shape, jnp.float32),
    )(x2, h, w)


@functools.lru_cache(maxsize=None)
def _sc_agg(nc, n_rows, zrow0):
    part = n_rows // 8
    acc_rows = part
    zpt = acc_rows // NT
    dpt = part // NT
    assert part % (NT * 8) == 0 and zpt % 8 == 0 and zpt <= ZPT
    mesh = plsc.VectorSubcoreMesh(core_axis_name="c", subcore_axis_name="s")

    @functools.partial(
        pl.kernel,
        out_type=jax.ShapeDtypeStruct((n_rows, D), jnp.float32),
        mesh=mesh,
        compiler_params=pltpu.CompilerParams(needs_layout_passes=False),
        scratch_types=[
            pltpu.VMEM((2, BCH, K), jnp.int32),
            pltpu.VMEM((2, BCH, K), jnp.int32),
            pltpu.VMEM((2, K, D), jnp.float32),
            pltpu.VMEM((ZPT, D), jnp.float32),
            pltpu.VMEM((ZPT, D), jnp.float32),
            pltpu.VMEM((2 * K,), jnp.int32),
            pltpu.VMEM((2 * K,), jnp.int32),
            pltpu.VMEM((2, K), jnp.int32),
            pltpu.VMEM((2, K), jnp.int32),
            pltpu.VMEM_SHARED((acc_rows, D), jnp.float32),
            pltpu.VMEM((16,), jnp.int32),
            pltpu.SemaphoreType.DMA,
            pltpu.SemaphoreType.DMA,
            pltpu.SemaphoreType.DMA,
        ],
    )
    def agg(x_hbm, src_hbm, dst_hbm, zero_hbm, ncs_hbm, out_hbm,
            sblk_v, dblk_v, rows_v, zbuf_v, stage_v, csrc_v, cdst_v,
            fsrc_v, fdst_v, acc, ncs_v, gsem, isem, ssem):
        cid = lax.axis_index("c")
        sid = lax.axis_index("s")
        pltpu.sync_copy(ncs_hbm, ncs_v)
        pltpu.sync_copy(zero_hbm, zbuf_v)
        mv = ncs_v[pl.ds(0, 16)]
        ncd = mv[0]
        it = mv[8]
        lane = lax.broadcasted_iota(jnp.int32, (16,), 0)

        def start_blk(buf, b0):
            pltpu.async_copy(src_hbm.at[it, sid, pl.ds(b0, BCH)],
                             sblk_v.at[buf], isem)
            pltpu.async_copy(dst_hbm.at[it, sid, pl.ds(b0, BCH)],
                             dblk_v.at[buf], isem)

        def wait_blk(buf, b0):
            pltpu.make_async_copy(src_hbm.at[it, sid, pl.ds(b0, BCH)],
                                  sblk_v.at[buf], isem).wait()
            pltpu.make_async_copy(dst_hbm.at[it, sid, pl.ds(b0, BCH)],
                                  dblk_v.at[buf], isem).wait()

        def wait_gather(b):
            pltpu.make_async_copy(x_hbm.at[fsrc_v.at[b]], rows_v.at[b],
                                  gsem).wait()

        def start_scatter(b):
            pltpu.async_copy(rows_v.at[b], acc.at[fdst_v.at[b]], ssem,
                             add=True)

        def wait_scatter(b):
            pltpu.make_async_copy(rows_v.at[b], acc.at[fdst_v.at[b]],
                                  ssem).wait()

        def flush_static(b, pend):
            @pl.when(pend >= 2)
            def _():
                wait_scatter(b)

            for j in range(K // 16):
                fsrc_v[b, pl.ds(16 * j, 16)] = csrc_v[pl.ds(16 * j, 16)]
                fdst_v[b, pl.ds(16 * j, 16)] = cdst_v[pl.ds(16 * j, 16)]
            @pl.when(pend >= 1)
            def _():
                wait_gather(1 - b)
                start_scatter(1 - b)

            pltpu.async_copy(x_hbm.at[fsrc_v.at[b]], rows_v.at[b], gsem)

        def flush(fb, pend, extra=None):
            for b in range(2):
                @pl.when(fb == b)
                def _(b=b):
                    flush_static(b, pend)
                    if extra is not None:
                        extra()

        def drain_all(fb, pend):
            for b in range(2):
                @pl.when(fb == b)
                def _(b=b):
                    wait_gather(b)
                    start_scatter(b)

                    @pl.when(pend >= 2)
                    def _():
                        wait_scatter(1 - b)

                    wait_scatter(b)

        def move_ring_down():
            for j in range(K // 16):
                csrc_v[pl.ds(16 * j, 16)] = csrc_v[pl.ds(K + 16 * j, 16)]
                cdst_v[pl.ds(16 * j, 16)] = cdst_v[pl.ds(K + 16 * j, 16)]

        z0 = sid * zpt
        pltpu.sync_copy(zbuf_v, acc.at[pl.ds(z0, zpt)])
        plsc.subcore_barrier()

        for p in range(4):
            lo = (4 * cid + p) * part

            def proc_block(buf, carry):
                def chunk(c, carry):
                    ptrv, fb, pend = carry
                    for j in range(K // 16):
                        d = dblk_v[buf, c, pl.ds(16 * j, 16)]
                        s = sblk_v[buf, c, pl.ds(16 * j, 16)]
                        dl = d - lo
                        m = dl.astype(jnp.uint32) < jnp.uint32(part)
                        m32 = m.astype(jnp.int32)
                        cs = plsc.cumsum(m32)
                        pos = ptrv + cs - m32
                        plsc.store_scatter(csrc_v, [pos], s, mask=m)
                        plsc.store_scatter(cdst_v, [pos], dl, mask=m)
                        ptrv = ptrv + plsc.all_reduce_population_count(m)
                    full = ptrv[0] >= K

                    @pl.when(full)
                    def _():
                        flush(fb, pend, extra=move_ring_down)

                    ptrv = jnp.where(full, ptrv - K, ptrv)
                    pend = jnp.where(full, jnp.minimum(pend + 1, 2), pend)
                    fb = jnp.where(full, 1 - fb, fb)
                    return ptrv, fb, pend

                return lax.fori_loop(0, BCH, chunk, carry, unroll=False)

            start_blk(0, 0)

            def blkpair(i, carry):
                b0 = 2 * i * BCH
                wait_blk(0, b0)
                start_blk(1, b0 + BCH)
                carry = proc_block(0, carry)
                wait_blk(1, b0 + BCH)

                @pl.when(b0 + 2 * BCH < ncd)
                def _():
                    start_blk(0, b0 + 2 * BCH)

                return proc_block(1, carry)

            ptrv, fb, pend = lax.fori_loop(
                0, ncd // (2 * BCH), blkpair,
                (jnp.zeros((16,), jnp.int32), jnp.int32(0), jnp.int32(0)),
                unroll=False)
            ptr = ptrv[0]
            for j in range(K // 16):
                idxv = ptr + 16 * j + lane
                plsc.store_scatter(csrc_v, [idxv],
                                   zrow0 + ((lane + 16 * j) & 15))
                plsc.store_scatter(cdst_v, [idxv],
                                   ((lane + 16 * j) * 8) & (part - 1))
            flush(fb, pend)
            pend = jnp.minimum(pend + 1, 2)
            drain_all(fb, pend)
            plsc.subcore_barrier()
            d0 = sid * dpt
            pltpu.sync_copy(acc.at[pl.ds(d0, dpt)], stage_v.at[pl.ds(0, dpt)])
            pltpu.sync_copy(stage_v.at[pl.ds(0, dpt)],
                            out_hbm.at[pl.ds(lo + d0, dpt)])
            if p < 3:
                pltpu.sync_copy(zbuf_v.at[pl.ds(0, dpt)],
                                acc.at[pl.ds(d0, dpt)])
            plsc.subcore_barrier()

    return agg


def _prep_edges(src, dst, n_nodes, ncmax):
    e = src.shape[0]
    per = NT * K
    nc = -(-(-(-e // per)) // (2 * BCH)) * (2 * BCH)
    pad = nc * per - e
    if pad:
        ar = jnp.arange(pad, dtype=jnp.int32)
        src = jnp.concatenate([src, (ar * 97) % n_nodes])
        dst = jnp.concatenate([dst, jnp.full((pad,), -1, jnp.int32)])
    src = src.reshape(NT, nc, K)
    dst = dst.reshape(NT, nc, K)
    if ncmax > nc:
        src = jnp.pad(src, ((0, 0), (0, ncmax - nc), (0, 0)))
        dst = jnp.pad(dst, ((0, 0), (0, ncmax - nc), (0, 0)), constant_values=-1)
    return src, dst, nc


def kernel(edge_index_drop, edge_index, features, preference, mlp_w, mlp_b,
           conv_w):
    n_user = preference.shape[0]
    n_nodes = n_user + features.shape[0]
    n_rows = -(-n_nodes // (4 * NT * 8)) * (4 * NT * 8)

    e1 = edge_index_drop.shape[1]
    e2 = 2 * edge_index.shape[1]
    ncmax = -(-(-(-max(e1, e2) // (NT * K))) // (2 * BCH)) * (2 * BCH)

    p0, fm = _tc_pre(preference, features, mlp_w, mlp_b)
    x0 = jnp.concatenate([p0, fm, jnp.zeros((16, D), jnp.float32)], axis=0)
    zeros = jnp.zeros((ZPT, D), jnp.float32)
    sc = _sc_agg(ncmax, n_rows, n_nodes)

    s1, d1, nc1 = _prep_edges(edge_index_drop[0], edge_index_drop[1],
                              n_nodes, ncmax)
    s2, d2, nc2 = _prep_edges(
        jnp.concatenate([edge_index[0], edge_index[1]]),
        jnp.concatenate([edge_index[1], edge_index[0]]),
        n_nodes, ncmax)
    src_st = jnp.stack([s1, s2])
    dst_st = jnp.stack([d1, d2])

    def body(i, carry):
        x, out = carry
        ncd = jnp.where(i == 0, nc1, nc2).astype(jnp.int32)
        meta = jnp.concatenate([jnp.full((8,), ncd, jnp.int32),
                                jnp.full((8,), i, jnp.int32)])
        h = sc(x, src_st, dst_st, zeros, meta)

        def mid(args):
            x, h = args
            pref = _tc_mid(x[:n_user], h[:n_user], conv_w)
            return jnp.concatenate([pref, x[n_user:]], axis=0), x[:n_nodes]

        def fin(args):
            x, h = args
            return x, _tc_fin(x[:n_nodes], h[:n_nodes], conv_w)

        return lax.cond(i == 0, mid, fin, (x, h))

    two = lax.optimization_barrier(jnp.int32(2))
    xf, out = lax.fori_loop(0, two, body, (x0, x0[:n_nodes]))
    return (out, xf[:n_user])
